# baseline (speedup 1.0000x reference)
import functools

import jax
import jax.numpy as jnp
from jax import lax
from jax.experimental import pallas as pl
from jax.experimental.pallas import tpu as pltpu
from jax.experimental.pallas import tpu_sc as plsc

_B = 4
_C = 64
_NY = 512
_NX = 512
_S = _NY * _NX
_N = _B * _S
_P = _B * 20000

_CH = 128
_NCHUNK = _P // _CH
_NW = 32
_ZB = 16384
_NBUF = 4

_SB = 4096


def _sc_scatter_body(feat_hbm, idx_hbm, canvas_hbm, mask_hbm,
                     idx_v, rows_v, ones_v, zeros_v,
                     load_sems, scat_sems):
    core = lax.axis_index("c")
    sub = lax.axis_index("s")
    wid = sub * 2 + core

    nk = (_NCHUNK + _NW - 1) // _NW

    def _cid(k):
        return jnp.minimum(wid + _NW * k, _NCHUNK - 1)

    def _start_load(k, buf):
        cid = _cid(k)
        pltpu.async_copy(idx_hbm.at[pl.ds(cid * _CH, _CH)],
                         idx_v.at[buf], load_sems.at[buf])
        pltpu.async_copy(feat_hbm.at[pl.ds(cid * _CH, _CH), :],
                         rows_v.at[buf], load_sems.at[buf])

    def _wait_load(k, buf):
        cid = _cid(k)
        pltpu.make_async_copy(idx_hbm.at[pl.ds(cid * _CH, _CH)],
                              idx_v.at[buf], load_sems.at[buf]).wait()
        pltpu.make_async_copy(feat_hbm.at[pl.ds(cid * _CH, _CH), :],
                              rows_v.at[buf], load_sems.at[buf]).wait()

    def _start_scatter(buf):
        pltpu.async_copy(rows_v.at[buf], canvas_hbm.at[idx_v.at[buf]],
                         scat_sems.at[buf])

    def _wait_scatter(buf):
        pltpu.make_async_copy(rows_v.at[buf], canvas_hbm.at[idx_v.at[0]],
                              scat_sems.at[buf]).wait()

    _start_load(0, 0)
    _start_load(1, 1)

    def _zfill(i, _):
        zeros_v[pl.ds(i * 16, 16)] = jnp.zeros((16,), jnp.int32)
        return ()
    lax.fori_loop(0, _ZB // 16, _zfill, ())

    @pl.when(core == 0)
    def _zero_mask():
        base = sub * (_N // 16)
        for j in range(_N // 16 // _ZB):
            pltpu.async_copy(zeros_v, mask_hbm.at[pl.ds(base + j * _ZB, _ZB)],
                             scat_sems.at[j])
        for j in range(_N // 16 // _ZB):
            pltpu.make_async_copy(
                zeros_v, mask_hbm.at[pl.ds(base + j * _ZB, _ZB)],
                scat_sems.at[j]).wait()

    plsc.subcore_barrier()

    for k in range(nk):
        buf = k % _NBUF
        _wait_load(k, buf)
        _start_scatter(buf)
        n = k + 2
        if n < nk:
            if n >= _NBUF:
                _wait_scatter(n % _NBUF)
            _start_load(n, n % _NBUF)
    for j in range(max(nk - _NBUF, 0), nk):
        _wait_scatter(j % _NBUF)

    ones_v[...] = jnp.ones((_CH,), jnp.int32)
    nk0 = (_NCHUNK + 15) // 16

    @pl.when(core == 0)
    def _scatter_ones():
        def _mcid(k):
            return jnp.minimum(sub + 16 * k, _NCHUNK - 1)

        def _mstart_load(k, buf):
            pltpu.async_copy(idx_hbm.at[pl.ds(_mcid(k) * _CH, _CH)],
                             idx_v.at[buf], load_sems.at[buf])

        def _mwait_load(k, buf):
            pltpu.make_async_copy(idx_hbm.at[pl.ds(_mcid(k) * _CH, _CH)],
                                  idx_v.at[buf], load_sems.at[buf]).wait()

        def _wait_ones(buf):
            pltpu.make_async_copy(ones_v, mask_hbm.at[idx_v.at[0]],
                                  scat_sems.at[buf]).wait()

        _mstart_load(0, 0)
        _mstart_load(1, 1)
        for k in range(nk0):
            buf = k % _NBUF
            _mwait_load(k, buf)
            pltpu.async_copy(ones_v, mask_hbm.at[idx_v.at[buf]],
                             scat_sems.at[buf])
            n = k + 2
            if n < nk0:
                if n >= _NBUF:
                    _wait_ones(n % _NBUF)
                _mstart_load(n, n % _NBUF)
        for j in range(max(nk0 - _NBUF, 0), nk0):
            _wait_ones(j % _NBUF)


_sc_scatter = functools.partial(
    pl.kernel,
    out_type=(
        jax.ShapeDtypeStruct((_N, 128), jnp.float32),
        jax.ShapeDtypeStruct((_N,), jnp.int32),
    ),
    mesh=plsc.VectorSubcoreMesh(core_axis_name="c", subcore_axis_name="s"),
    compiler_params=pltpu.CompilerParams(use_tc_tiling_on_sc=True),
    scratch_types=[
        pltpu.VMEM((_NBUF, _CH), jnp.int32),
        pltpu.VMEM((_NBUF, _CH, 128), jnp.float32),
        pltpu.VMEM((_CH,), jnp.int32),
        pltpu.VMEM((_ZB,), jnp.int32),
        pltpu.SemaphoreType.DMA((_NBUF,)),
        pltpu.SemaphoreType.DMA((_NBUF,)),
    ],
)(_sc_scatter_body)


def _pad_body(x_ref, o_ref):
    o_ref[:, :_C] = x_ref[...]


def _pad128(x):
    return pl.pallas_call(
        _pad_body,
        grid=(_P // 2000,),
        in_specs=[pl.BlockSpec((2000, _C), lambda i: (i, 0))],
        out_specs=pl.BlockSpec((2000, 128), lambda i: (i, 0)),
        out_shape=jax.ShapeDtypeStruct((_P, 128), jnp.float32),
    )(x)


def _tc_transpose_body(canvas_ref, mask_ref, out_ref):
    x = canvas_ref[0][:, :_C]
    m2 = mask_ref[...]
    m = m2.reshape(1, _SB)
    eye = (lax.broadcasted_iota(jnp.int32, (_C, _C), 0)
           == lax.broadcasted_iota(jnp.int32, (_C, _C), 1)
           ).astype(jnp.float32)
    xt = lax.dot_general(eye, x, (((1,), (1,)), ((), ())),
                         preferred_element_type=jnp.float32)
    sel = jnp.where(m != 0, xt, jnp.float32(0))
    out_ref[0] = sel.reshape(_C, _SB // _NX, _NX)


def _tc_transpose(canvas, mask):
    grid = (_B, _S // _SB)
    return pl.pallas_call(
        _tc_transpose_body,
        grid=grid,
        in_specs=[
            pl.BlockSpec((1, _SB, 128), lambda b, s: (b, s, 0)),
            pl.BlockSpec((_SB // 128, 128), lambda b, s: (b * (_S // _SB) + s, 0)),
        ],
        out_specs=pl.BlockSpec((1, _C, _SB // _NX, _NX), lambda b, s: (b, 0, s, 0)),
        out_shape=jax.ShapeDtypeStruct((_B, _C, _NY, _NX), jnp.float32),
    )(canvas, mask)


def kernel(pillar_features, voxel_coords):
    coords = voxel_coords.astype(jnp.int32)
    flat_idx = coords[:, 0] * _S + coords[:, 2] * _NX + coords[:, 3]
    feat128 = _pad128(pillar_features)
    canvas, mask = _sc_scatter(feat128, flat_idx)
    return _tc_transpose(canvas.reshape(_B, _S, 128),
                         mask.reshape(_N // 128, 128))


# speedup vs baseline: 1.1660x; 1.1660x over previous
import functools

import jax
import jax.numpy as jnp
from jax import lax
from jax.experimental import pallas as pl
from jax.experimental.pallas import tpu as pltpu
from jax.experimental.pallas import tpu_sc as plsc

_B = 4
_C = 64
_NY = 512
_NX = 512
_S = _NY * _NX
_N = _B * _S
_P = _B * 20000

_CH = 128
_NCHUNK = _P // _CH
_NW = 32
_ZB = 16384
_NBUF = 4

_SB = 8192


def _sc_scatter_body(feat_hbm, idx_hbm, canvas_hbm, mask_hbm,
                     idx_v, rows_v, ones_v, zeros_v,
                     load_sems, scat_sems):
    core = lax.axis_index("c")
    sub = lax.axis_index("s")
    wid = sub * 2 + core

    nk = (_NCHUNK + _NW - 1) // _NW

    def _cid(k):
        return jnp.minimum(wid + _NW * k, _NCHUNK - 1)

    def _start_load(k, buf):
        cid = _cid(k)
        pltpu.async_copy(idx_hbm.at[pl.ds(cid * _CH, _CH)],
                         idx_v.at[buf], load_sems.at[buf])
        pltpu.async_copy(feat_hbm.at[pl.ds(cid * _CH, _CH), :],
                         rows_v.at[buf], load_sems.at[buf])

    def _wait_load(k, buf):
        cid = _cid(k)
        pltpu.make_async_copy(idx_hbm.at[pl.ds(cid * _CH, _CH)],
                              idx_v.at[buf], load_sems.at[buf]).wait()
        pltpu.make_async_copy(feat_hbm.at[pl.ds(cid * _CH, _CH), :],
                              rows_v.at[buf], load_sems.at[buf]).wait()

    def _start_scatter(buf):
        pltpu.async_copy(rows_v.at[buf], canvas_hbm.at[idx_v.at[buf]],
                         scat_sems.at[buf])

    def _wait_scatter(buf):
        pltpu.make_async_copy(rows_v.at[buf], canvas_hbm.at[idx_v.at[0]],
                              scat_sems.at[buf]).wait()

    _start_load(0, 0)
    _start_load(1, 1)

    def _zfill(i, _):
        zeros_v[pl.ds(i * 16, 16)] = jnp.zeros((16,), jnp.int32)
        return ()
    lax.fori_loop(0, _ZB // 16, _zfill, ())

    @pl.when(core == 0)
    def _zero_mask():
        base = sub * (_N // 16)
        for j in range(_N // 16 // _ZB):
            pltpu.async_copy(zeros_v, mask_hbm.at[pl.ds(base + j * _ZB, _ZB)],
                             scat_sems.at[j])
        for j in range(_N // 16 // _ZB):
            pltpu.make_async_copy(
                zeros_v, mask_hbm.at[pl.ds(base + j * _ZB, _ZB)],
                scat_sems.at[j]).wait()

    plsc.subcore_barrier()

    for k in range(nk):
        buf = k % _NBUF
        _wait_load(k, buf)
        _start_scatter(buf)
        n = k + 2
        if n < nk:
            if n >= _NBUF:
                _wait_scatter(n % _NBUF)
            _start_load(n, n % _NBUF)
    for j in range(max(nk - _NBUF, 0), nk):
        _wait_scatter(j % _NBUF)

    ones_v[...] = jnp.ones((_CH,), jnp.int32)
    nk0 = (_NCHUNK + 15) // 16

    @pl.when(core == 0)
    def _scatter_ones():
        def _mcid(k):
            return jnp.minimum(sub + 16 * k, _NCHUNK - 1)

        def _mstart_load(k, buf):
            pltpu.async_copy(idx_hbm.at[pl.ds(_mcid(k) * _CH, _CH)],
                             idx_v.at[buf], load_sems.at[buf])

        def _mwait_load(k, buf):
            pltpu.make_async_copy(idx_hbm.at[pl.ds(_mcid(k) * _CH, _CH)],
                                  idx_v.at[buf], load_sems.at[buf]).wait()

        def _wait_ones(buf):
            pltpu.make_async_copy(ones_v, mask_hbm.at[idx_v.at[0]],
                                  scat_sems.at[buf]).wait()

        _mstart_load(0, 0)
        _mstart_load(1, 1)
        for k in range(nk0):
            buf = k % _NBUF
            _mwait_load(k, buf)
            pltpu.async_copy(ones_v, mask_hbm.at[idx_v.at[buf]],
                             scat_sems.at[buf])
            n = k + 2
            if n < nk0:
                if n >= _NBUF:
                    _wait_ones(n % _NBUF)
                _mstart_load(n, n % _NBUF)
        for j in range(max(nk0 - _NBUF, 0), nk0):
            _wait_ones(j % _NBUF)


_sc_scatter = functools.partial(
    pl.kernel,
    out_type=(
        jax.ShapeDtypeStruct((_N, 128), jnp.float32),
        jax.ShapeDtypeStruct((_N,), jnp.int32),
    ),
    mesh=plsc.VectorSubcoreMesh(core_axis_name="c", subcore_axis_name="s"),
    compiler_params=pltpu.CompilerParams(use_tc_tiling_on_sc=True),
    scratch_types=[
        pltpu.VMEM((_NBUF, _CH), jnp.int32),
        pltpu.VMEM((_NBUF, _CH, 128), jnp.float32),
        pltpu.VMEM((_CH,), jnp.int32),
        pltpu.VMEM((_ZB,), jnp.int32),
        pltpu.SemaphoreType.DMA((_NBUF,)),
        pltpu.SemaphoreType.DMA((_NBUF,)),
    ],
)(_sc_scatter_body)


def _pad_body(x_ref, o_ref):
    o_ref[:, :_C] = x_ref[...]


def _pad128(x):
    return pl.pallas_call(
        _pad_body,
        grid=(_P // 2000,),
        in_specs=[pl.BlockSpec((2000, _C), lambda i: (i, 0))],
        out_specs=pl.BlockSpec((2000, 128), lambda i: (i, 0)),
        out_shape=jax.ShapeDtypeStruct((_P, 128), jnp.float32),
    )(x)


def _tc_transpose_body(canvas_ref, mask_ref, out_ref):
    x = canvas_ref[0][:, :_C]
    m2 = mask_ref[...]
    m = m2.reshape(1, _SB)
    eye = (lax.broadcasted_iota(jnp.int32, (_C, _C), 0)
           == lax.broadcasted_iota(jnp.int32, (_C, _C), 1)
           ).astype(jnp.float32)
    xt = lax.dot_general(eye, x, (((1,), (1,)), ((), ())),
                         preferred_element_type=jnp.float32)
    sel = jnp.where(m != 0, xt, jnp.float32(0))
    out_ref[0] = sel.reshape(_C, _SB // _NX, _NX)


def _tc_transpose(canvas, mask):
    grid = (_B, _S // _SB)
    return pl.pallas_call(
        _tc_transpose_body,
        grid=grid,
        in_specs=[
            pl.BlockSpec((1, _SB, 128), lambda b, s: (b, s, 0)),
            pl.BlockSpec((_SB // 128, 128), lambda b, s: (b * (_S // _SB) + s, 0)),
        ],
        out_specs=pl.BlockSpec((1, _C, _SB // _NX, _NX), lambda b, s: (b, 0, s, 0)),
        out_shape=jax.ShapeDtypeStruct((_B, _C, _NY, _NX), jnp.float32),
    )(canvas, mask)


def kernel(pillar_features, voxel_coords):
    coords = voxel_coords.astype(jnp.int32)
    flat_idx = coords[:, 0] * _S + coords[:, 2] * _NX + coords[:, 3]
    feat128 = _pad128(pillar_features)
    canvas, mask = _sc_scatter(feat128, flat_idx)
    return _tc_transpose(canvas.reshape(_B, _S, 128),
                         mask.reshape(_N // 128, 128))


# speedup vs baseline: 1.2160x; 1.0429x over previous
import functools

import jax
import jax.numpy as jnp
from jax import lax
from jax.experimental import pallas as pl
from jax.experimental.pallas import tpu as pltpu
from jax.experimental.pallas import tpu_sc as plsc

_B = 4
_C = 64
_NY = 512
_NX = 512
_S = _NY * _NX
_N = _B * _S
_P = _B * 20000

_CH = 128
_NCHUNK = _P // _CH
_NW = 32
_ZB = 16384
_NBUF = 4

_SB = 16384


def _sc_scatter_body(feat_hbm, idx_hbm, canvas_hbm, mask_hbm,
                     idx_v, rows_v, ones_v, zeros_v,
                     load_sems, scat_sems):
    core = lax.axis_index("c")
    sub = lax.axis_index("s")
    wid = sub * 2 + core

    nk = (_NCHUNK + _NW - 1) // _NW

    def _cid(k):
        return jnp.minimum(wid + _NW * k, _NCHUNK - 1)

    def _start_load(k, buf):
        cid = _cid(k)
        pltpu.async_copy(idx_hbm.at[pl.ds(cid * _CH, _CH)],
                         idx_v.at[buf], load_sems.at[buf])
        pltpu.async_copy(feat_hbm.at[pl.ds(cid * _CH, _CH), :],
                         rows_v.at[buf], load_sems.at[buf])

    def _wait_load(k, buf):
        cid = _cid(k)
        pltpu.make_async_copy(idx_hbm.at[pl.ds(cid * _CH, _CH)],
                              idx_v.at[buf], load_sems.at[buf]).wait()
        pltpu.make_async_copy(feat_hbm.at[pl.ds(cid * _CH, _CH), :],
                              rows_v.at[buf], load_sems.at[buf]).wait()

    def _start_scatter(buf):
        pltpu.async_copy(rows_v.at[buf], canvas_hbm.at[idx_v.at[buf]],
                         scat_sems.at[buf])

    def _wait_scatter(buf):
        pltpu.make_async_copy(rows_v.at[buf], canvas_hbm.at[idx_v.at[0]],
                              scat_sems.at[buf]).wait()

    _start_load(0, 0)
    _start_load(1, 1)

    def _zfill(i, _):
        zeros_v[pl.ds(i * 16, 16)] = jnp.zeros((16,), jnp.int32)
        return ()
    lax.fori_loop(0, _ZB // 16, _zfill, ())

    @pl.when(core == 0)
    def _zero_mask():
        base = sub * (_N // 16)
        for j in range(_N // 16 // _ZB):
            pltpu.async_copy(zeros_v, mask_hbm.at[pl.ds(base + j * _ZB, _ZB)],
                             scat_sems.at[j])
        for j in range(_N // 16 // _ZB):
            pltpu.make_async_copy(
                zeros_v, mask_hbm.at[pl.ds(base + j * _ZB, _ZB)],
                scat_sems.at[j]).wait()

    plsc.subcore_barrier()

    for k in range(nk):
        buf = k % _NBUF
        _wait_load(k, buf)
        _start_scatter(buf)
        n = k + 2
        if n < nk:
            if n >= _NBUF:
                _wait_scatter(n % _NBUF)
            _start_load(n, n % _NBUF)
    for j in range(max(nk - _NBUF, 0), nk):
        _wait_scatter(j % _NBUF)

    ones_v[...] = jnp.ones((_CH,), jnp.int32)
    nk0 = (_NCHUNK + 15) // 16

    @pl.when(core == 0)
    def _scatter_ones():
        def _mcid(k):
            return jnp.minimum(sub + 16 * k, _NCHUNK - 1)

        def _mstart_load(k, buf):
            pltpu.async_copy(idx_hbm.at[pl.ds(_mcid(k) * _CH, _CH)],
                             idx_v.at[buf], load_sems.at[buf])

        def _mwait_load(k, buf):
            pltpu.make_async_copy(idx_hbm.at[pl.ds(_mcid(k) * _CH, _CH)],
                                  idx_v.at[buf], load_sems.at[buf]).wait()

        def _wait_ones(buf):
            pltpu.make_async_copy(ones_v, mask_hbm.at[idx_v.at[0]],
                                  scat_sems.at[buf]).wait()

        _mstart_load(0, 0)
        _mstart_load(1, 1)
        for k in range(nk0):
            buf = k % _NBUF
            _mwait_load(k, buf)
            pltpu.async_copy(ones_v, mask_hbm.at[idx_v.at[buf]],
                             scat_sems.at[buf])
            n = k + 2
            if n < nk0:
                if n >= _NBUF:
                    _wait_ones(n % _NBUF)
                _mstart_load(n, n % _NBUF)
        for j in range(max(nk0 - _NBUF, 0), nk0):
            _wait_ones(j % _NBUF)


_sc_scatter = functools.partial(
    pl.kernel,
    out_type=(
        jax.ShapeDtypeStruct((_N, 128), jnp.float32),
        jax.ShapeDtypeStruct((_N,), jnp.int32),
    ),
    mesh=plsc.VectorSubcoreMesh(core_axis_name="c", subcore_axis_name="s"),
    compiler_params=pltpu.CompilerParams(use_tc_tiling_on_sc=True),
    scratch_types=[
        pltpu.VMEM((_NBUF, _CH), jnp.int32),
        pltpu.VMEM((_NBUF, _CH, 128), jnp.float32),
        pltpu.VMEM((_CH,), jnp.int32),
        pltpu.VMEM((_ZB,), jnp.int32),
        pltpu.SemaphoreType.DMA((_NBUF,)),
        pltpu.SemaphoreType.DMA((_NBUF,)),
    ],
)(_sc_scatter_body)


def _pad_body(x_ref, o_ref):
    o_ref[:, :_C] = x_ref[...]


def _pad128(x):
    return pl.pallas_call(
        _pad_body,
        grid=(_P // 2000,),
        in_specs=[pl.BlockSpec((2000, _C), lambda i: (i, 0))],
        out_specs=pl.BlockSpec((2000, 128), lambda i: (i, 0)),
        out_shape=jax.ShapeDtypeStruct((_P, 128), jnp.float32),
    )(x)


def _tc_transpose_body(canvas_ref, mask_ref, out_ref):
    x = canvas_ref[0][:, :_C]
    m2 = mask_ref[...]
    m = m2.reshape(1, _SB)
    eye = (lax.broadcasted_iota(jnp.int32, (_C, _C), 0)
           == lax.broadcasted_iota(jnp.int32, (_C, _C), 1)
           ).astype(jnp.float32)
    xt = lax.dot_general(eye, x, (((1,), (1,)), ((), ())),
                         preferred_element_type=jnp.float32)
    sel = jnp.where(m != 0, xt, jnp.float32(0))
    out_ref[0] = sel.reshape(_C, _SB // _NX, _NX)


def _tc_transpose(canvas, mask):
    grid = (_B, _S // _SB)
    return pl.pallas_call(
        _tc_transpose_body,
        grid=grid,
        in_specs=[
            pl.BlockSpec((1, _SB, 128), lambda b, s: (b, s, 0)),
            pl.BlockSpec((_SB // 128, 128), lambda b, s: (b * (_S // _SB) + s, 0)),
        ],
        out_specs=pl.BlockSpec((1, _C, _SB // _NX, _NX), lambda b, s: (b, 0, s, 0)),
        out_shape=jax.ShapeDtypeStruct((_B, _C, _NY, _NX), jnp.float32),
    )(canvas, mask)


def kernel(pillar_features, voxel_coords):
    coords = voxel_coords.astype(jnp.int32)
    flat_idx = coords[:, 0] * _S + coords[:, 2] * _NX + coords[:, 3]
    feat128 = _pad128(pillar_features)
    canvas, mask = _sc_scatter(feat128, flat_idx)
    return _tc_transpose(canvas.reshape(_B, _S, 128),
                         mask.reshape(_N // 128, 128))


# speedup vs baseline: 1.2329x; 1.0139x over previous
import functools

import jax
import jax.numpy as jnp
from jax import lax
from jax.experimental import pallas as pl
from jax.experimental.pallas import tpu as pltpu
from jax.experimental.pallas import tpu_sc as plsc

_B = 4
_C = 64
_NY = 512
_NX = 512
_S = _NY * _NX
_N = _B * _S
_P = _B * 20000

_CH = 128
_NCHUNK = _P // _CH
_NW = 32
_ZB = 16384
_NBUF = 4

_SB = 32768


def _sc_scatter_body(feat_hbm, idx_hbm, canvas_hbm, mask_hbm,
                     idx_v, rows_v, ones_v, zeros_v,
                     load_sems, scat_sems):
    core = lax.axis_index("c")
    sub = lax.axis_index("s")
    wid = sub * 2 + core

    nk = (_NCHUNK + _NW - 1) // _NW

    def _cid(k):
        return jnp.minimum(wid + _NW * k, _NCHUNK - 1)

    def _start_load(k, buf):
        cid = _cid(k)
        pltpu.async_copy(idx_hbm.at[pl.ds(cid * _CH, _CH)],
                         idx_v.at[buf], load_sems.at[buf])
        pltpu.async_copy(feat_hbm.at[pl.ds(cid * _CH, _CH), :],
                         rows_v.at[buf], load_sems.at[buf])

    def _wait_load(k, buf):
        cid = _cid(k)
        pltpu.make_async_copy(idx_hbm.at[pl.ds(cid * _CH, _CH)],
                              idx_v.at[buf], load_sems.at[buf]).wait()
        pltpu.make_async_copy(feat_hbm.at[pl.ds(cid * _CH, _CH), :],
                              rows_v.at[buf], load_sems.at[buf]).wait()

    def _start_scatter(buf):
        pltpu.async_copy(rows_v.at[buf], canvas_hbm.at[idx_v.at[buf]],
                         scat_sems.at[buf])

    def _wait_scatter(buf):
        pltpu.make_async_copy(rows_v.at[buf], canvas_hbm.at[idx_v.at[0]],
                              scat_sems.at[buf]).wait()

    _start_load(0, 0)
    _start_load(1, 1)

    def _zfill(i, _):
        zeros_v[pl.ds(i * 16, 16)] = jnp.zeros((16,), jnp.int32)
        return ()
    lax.fori_loop(0, _ZB // 16, _zfill, ())

    @pl.when(core == 0)
    def _zero_mask():
        base = sub * (_N // 16)
        for j in range(_N // 16 // _ZB):
            pltpu.async_copy(zeros_v, mask_hbm.at[pl.ds(base + j * _ZB, _ZB)],
                             scat_sems.at[j])
        for j in range(_N // 16 // _ZB):
            pltpu.make_async_copy(
                zeros_v, mask_hbm.at[pl.ds(base + j * _ZB, _ZB)],
                scat_sems.at[j]).wait()

    plsc.subcore_barrier()

    for k in range(nk):
        buf = k % _NBUF
        _wait_load(k, buf)
        _start_scatter(buf)
        n = k + 2
        if n < nk:
            if n >= _NBUF:
                _wait_scatter(n % _NBUF)
            _start_load(n, n % _NBUF)
    for j in range(max(nk - _NBUF, 0), nk):
        _wait_scatter(j % _NBUF)

    ones_v[...] = jnp.ones((_CH,), jnp.int32)
    nk0 = (_NCHUNK + 15) // 16

    @pl.when(core == 0)
    def _scatter_ones():
        def _mcid(k):
            return jnp.minimum(sub + 16 * k, _NCHUNK - 1)

        def _mstart_load(k, buf):
            pltpu.async_copy(idx_hbm.at[pl.ds(_mcid(k) * _CH, _CH)],
                             idx_v.at[buf], load_sems.at[buf])

        def _mwait_load(k, buf):
            pltpu.make_async_copy(idx_hbm.at[pl.ds(_mcid(k) * _CH, _CH)],
                                  idx_v.at[buf], load_sems.at[buf]).wait()

        def _wait_ones(buf):
            pltpu.make_async_copy(ones_v, mask_hbm.at[idx_v.at[0]],
                                  scat_sems.at[buf]).wait()

        _mstart_load(0, 0)
        _mstart_load(1, 1)
        for k in range(nk0):
            buf = k % _NBUF
            _mwait_load(k, buf)
            pltpu.async_copy(ones_v, mask_hbm.at[idx_v.at[buf]],
                             scat_sems.at[buf])
            n = k + 2
            if n < nk0:
                if n >= _NBUF:
                    _wait_ones(n % _NBUF)
                _mstart_load(n, n % _NBUF)
        for j in range(max(nk0 - _NBUF, 0), nk0):
            _wait_ones(j % _NBUF)


_sc_scatter = functools.partial(
    pl.kernel,
    out_type=(
        jax.ShapeDtypeStruct((_N, 128), jnp.float32),
        jax.ShapeDtypeStruct((_N,), jnp.int32),
    ),
    mesh=plsc.VectorSubcoreMesh(core_axis_name="c", subcore_axis_name="s"),
    compiler_params=pltpu.CompilerParams(use_tc_tiling_on_sc=True),
    scratch_types=[
        pltpu.VMEM((_NBUF, _CH), jnp.int32),
        pltpu.VMEM((_NBUF, _CH, 128), jnp.float32),
        pltpu.VMEM((_CH,), jnp.int32),
        pltpu.VMEM((_ZB,), jnp.int32),
        pltpu.SemaphoreType.DMA((_NBUF,)),
        pltpu.SemaphoreType.DMA((_NBUF,)),
    ],
)(_sc_scatter_body)


def _pad_body(x_ref, o_ref):
    o_ref[:, :_C] = x_ref[...]


def _pad128(x):
    return pl.pallas_call(
        _pad_body,
        grid=(_P // 2000,),
        in_specs=[pl.BlockSpec((2000, _C), lambda i: (i, 0))],
        out_specs=pl.BlockSpec((2000, 128), lambda i: (i, 0)),
        out_shape=jax.ShapeDtypeStruct((_P, 128), jnp.float32),
    )(x)


def _tc_transpose_body(canvas_ref, mask_ref, out_ref):
    x = canvas_ref[0][:, :_C]
    m2 = mask_ref[...]
    m = m2.reshape(1, _SB)
    eye = (lax.broadcasted_iota(jnp.int32, (_C, _C), 0)
           == lax.broadcasted_iota(jnp.int32, (_C, _C), 1)
           ).astype(jnp.float32)
    xt = lax.dot_general(eye, x, (((1,), (1,)), ((), ())),
                         preferred_element_type=jnp.float32)
    sel = jnp.where(m != 0, xt, jnp.float32(0))
    out_ref[0] = sel.reshape(_C, _SB // _NX, _NX)


def _tc_transpose(canvas, mask):
    grid = (_B, _S // _SB)
    return pl.pallas_call(
        _tc_transpose_body,
        grid=grid,
        in_specs=[
            pl.BlockSpec((1, _SB, 128), lambda b, s: (b, s, 0)),
            pl.BlockSpec((_SB // 128, 128), lambda b, s: (b * (_S // _SB) + s, 0)),
        ],
        out_specs=pl.BlockSpec((1, _C, _SB // _NX, _NX), lambda b, s: (b, 0, s, 0)),
        out_shape=jax.ShapeDtypeStruct((_B, _C, _NY, _NX), jnp.float32),
    )(canvas, mask)


def kernel(pillar_features, voxel_coords):
    coords = voxel_coords.astype(jnp.int32)
    flat_idx = coords[:, 0] * _S + coords[:, 2] * _NX + coords[:, 3]
    feat128 = _pad128(pillar_features)
    canvas, mask = _sc_scatter(feat128, flat_idx)
    return _tc_transpose(canvas.reshape(_B, _S, 128),
                         mask.reshape(_N // 128, 128))
